# baseline (speedup 1.0000x reference)
import dataclasses
import functools

import jax
import jax.numpy as jnp
from jax import lax
from jax.experimental import pallas as pl
from jax.experimental.pallas import tpu as pltpu
from jax.experimental.pallas import tpu_sc as plsc

N_NODES = 10000
N_EDGES = 320000
D = 128

NC = 2
NS = 16
NW = NC * NS
EDGES_PER_W = N_EDGES // NW
BLK = 64
N_PAD = 10240
ROWS_PER_TILE = N_PAD // NS
PADE = N_PAD - EDGES_PER_W
NBLK = N_PAD // BLK

_mesh = lambda: plsc.VectorSubcoreMesh(core_axis_name="c", subcore_axis_name="s")


def _pad_edges(src, dst):
    srcw = src.reshape(NW, EDGES_PER_W)
    dstw = dst.reshape(NW, EDGES_PER_W)
    ar = jnp.arange(PADE, dtype=jnp.int32)
    wid = jnp.arange(NW, dtype=jnp.int32)[:, None]
    pad_s = jnp.broadcast_to((ar * 41) % N_NODES, (NW, PADE))
    pad_h = N_NODES + (ar[None, :] + wid * 7) % PADE
    src_p = jnp.concatenate([srcw, pad_s], axis=1)
    dst_p = jnp.concatenate([dstw, pad_h], axis=1).reshape(NW, NBLK, BLK)
    return src_p, dst_p


def _sc_message_pass(table, src_p, dst_p):
    @functools.partial(
        pl.kernel,
        out_type=jax.ShapeDtypeStruct((NC, N_PAD, D), jnp.float32),
        mesh=_mesh(),
        scratch_types=[
            pltpu.VMEM((N_PAD,), jnp.int32),
            pltpu.VMEM((BLK,), jnp.int32),
            pltpu.VMEM((BLK,), jnp.int32),
            pltpu.VMEM((BLK,), jnp.int32),
            pltpu.VMEM((BLK,), jnp.int32),
            pltpu.VMEM((BLK, D), jnp.float32),
            pltpu.VMEM((BLK, D), jnp.float32),
            pltpu.VMEM((BLK, D), jnp.float32),
            pltpu.VMEM((BLK, D), jnp.float32),
            pltpu.VMEM((32, D), jnp.float32),
            pltpu.VMEM_SHARED((N_PAD, D), jnp.float32),
            pltpu.SemaphoreType.DMA,
            pltpu.SemaphoreType.DMA,
            pltpu.SemaphoreType.DMA,
            pltpu.SemaphoreType.DMA,
            pltpu.SemaphoreType.DMA,
            pltpu.SemaphoreType.DMA,
            pltpu.SemaphoreType.DMA,
            pltpu.SemaphoreType.DMA,
            pltpu.SemaphoreType.DMA,
            pltpu.SemaphoreType.DMA,
        ],
    )
    def k(table_hbm, srcp_hbm, dstp_hbm, out_hbm,
          sidx, didx0, didx1, didx2, didx3, rows0, rows1, rows2, rows3,
          zbuf, acc, gs0, gs1, gs2, gs3, ss0, ss1, ss2, ss3, isem, zsem):
        cid = lax.axis_index("c")
        sid = lax.axis_index("s")
        wid = sid * NC + cid
        my_row0 = sid * ROWS_PER_TILE
        my_dst = dstp_hbm.at[wid]
        didx = [didx0, didx1, didx2, didx3]
        rows = [rows0, rows1, rows2, rows3]
        gs = [gs0, gs1, gs2, gs3]
        ss = [ss0, ss1, ss2, ss3]

        pltpu.async_copy(srcp_hbm.at[wid], sidx, isem)

        @pl.loop(0, 32)
        def _(i):
            @pl.loop(0, D, step=16)
            def _(j):
                zbuf[i, pl.ds(j, 16)] = jnp.zeros((16,), jnp.float32)
        @pl.loop(0, ROWS_PER_TILE, step=32)
        def _(r):
            pltpu.async_copy(zbuf, acc.at[pl.ds(my_row0 + r, 32)], zsem)

        pltpu.make_async_copy(srcp_hbm.at[wid], sidx, isem).wait()
        pltpu.sync_copy(my_dst.at[0], didx0)
        pltpu.sync_copy(my_dst.at[1], didx1)
        pltpu.async_copy(table_hbm.at[sidx.at[pl.ds(0, BLK)]], rows0, gs0)
        pltpu.async_copy(table_hbm.at[sidx.at[pl.ds(BLK, BLK)]], rows1, gs1)

        @pl.loop(0, ROWS_PER_TILE, step=32)
        def _(r):
            pltpu.make_async_copy(zbuf, acc.at[pl.ds(my_row0 + r, 32)], zsem).wait()
        plsc.subcore_barrier()

        @pl.loop(0, NBLK, step=4)
        def _(b):
            for kk in range(4):
                bk = b + kk
                j = (kk + 2) % 4

                @pl.when(bk >= 2)
                def _():
                    pltpu.make_async_copy(rows[j], acc.at[didx[j]], ss[j]).wait()

                @pl.when(bk + 2 < NBLK)
                def _():
                    pltpu.sync_copy(my_dst.at[bk + 2], didx[j])

                pltpu.make_async_copy(table_hbm.at[sidx.at[pl.ds(bk * BLK, BLK)]],
                                      rows[kk], gs[kk]).wait()

                @pl.when(bk + 2 < NBLK)
                def _():
                    pltpu.async_copy(
                        table_hbm.at[sidx.at[pl.ds((bk + 2) * BLK, BLK)]],
                        rows[j], gs[j])

                pltpu.make_async_copy(rows[kk], acc.at[didx[kk]],
                                      ss[kk]).start(add=True)

        pltpu.make_async_copy(rows2, acc.at[didx2], ss2).wait()
        pltpu.make_async_copy(rows3, acc.at[didx3], ss3).wait()
        plsc.subcore_barrier()

        pltpu.sync_copy(acc.at[pl.ds(my_row0, ROWS_PER_TILE)],
                        out_hbm.at[cid].at[pl.ds(my_row0, ROWS_PER_TILE)])

    return k(table, src_p, dst_p)


def _sc_degrees(srcw, dstw):
    cp = pltpu.CompilerParams()
    if "needs_layout_passes" in pltpu.CompilerParams.__dataclass_fields__:
        cp = dataclasses.replace(cp, needs_layout_passes=False)

    @functools.partial(
        pl.kernel,
        out_type=jax.ShapeDtypeStruct((2, NW, N_PAD), jnp.float32),
        mesh=_mesh(),
        compiler_params=cp,
        scratch_types=[
            pltpu.VMEM((EDGES_PER_W,), jnp.int32),
            pltpu.VMEM((EDGES_PER_W,), jnp.int32),
            pltpu.VMEM((N_PAD,), jnp.float32),
            pltpu.VMEM((N_PAD,), jnp.float32),
            pltpu.SemaphoreType.DMA,
        ],
    )
    def k(src_hbm, dst_hbm, out_hbm, sidx, didx, hist_s, hist_d, isem):
        cid = lax.axis_index("c")
        sid = lax.axis_index("s")
        wid = sid * NC + cid

        pltpu.async_copy(src_hbm.at[wid], sidx, isem)
        pltpu.async_copy(dst_hbm.at[wid], didx, isem)

        zeros = jnp.zeros((16,), jnp.float32)
        @pl.loop(0, N_PAD, step=16)
        def _(j):
            hist_s[pl.ds(j, 16)] = zeros
            hist_d[pl.ds(j, 16)] = zeros

        pltpu.make_async_copy(src_hbm.at[wid], sidx, isem).wait()
        pltpu.make_async_copy(dst_hbm.at[wid], didx, isem).wait()

        ones = jnp.ones((16,), jnp.float32)
        @pl.loop(0, EDGES_PER_W, step=16)
        def _(e):
            plsc.addupdate_scatter(hist_s, [sidx[pl.ds(e, 16)]], ones)
            plsc.addupdate_scatter(hist_d, [didx[pl.ds(e, 16)]], ones)

        pltpu.sync_copy(hist_s, out_hbm.at[0].at[wid])
        pltpu.sync_copy(hist_d, out_hbm.at[1].at[wid])

    return k(srcw, dstw)


def _norms(d_ref):
    deg_out = jnp.sum(d_ref[:, :NW], axis=1, keepdims=True)
    deg_in = jnp.sum(d_ref[:, NW:], axis=1, keepdims=True)
    return (lax.rsqrt(jnp.maximum(deg_out, 1.0)),
            lax.rsqrt(jnp.maximum(deg_in, 1.0)))


def _tc_matmul_scale(x, w, degp_t):
    def body(x_ref, w_ref, d_ref, o_ref):
        ns, _ = _norms(d_ref)
        o_ref[...] = jnp.dot(x_ref[...], w_ref[...],
                             preferred_element_type=jnp.float32) * ns
    return pl.pallas_call(
        body, out_shape=jax.ShapeDtypeStruct((x.shape[0], w.shape[1]), jnp.float32),
    )(x, w, degp_t)


def _tc_mid(partials, degp_t, b1, w2):
    def body(p_ref, d_ref, b_ref, w_ref, o_ref):
        ns, nd = _norms(d_ref)
        u = (p_ref[0, :N_NODES] + p_ref[1, :N_NODES]) * nd + b_ref[...]
        u = jnp.maximum(u, 0.0)
        o_ref[...] = jnp.dot(u, w_ref[...],
                             preferred_element_type=jnp.float32) * ns
    return pl.pallas_call(
        body, out_shape=jax.ShapeDtypeStruct((N_NODES, D), jnp.float32),
    )(partials, degp_t, b1, w2)


def _tc_final(partials, degp_t, b2):
    def body(p_ref, d_ref, b_ref, o_ref):
        _, nd = _norms(d_ref)
        o_ref[...] = (p_ref[0, :N_NODES] + p_ref[1, :N_NODES]) * nd + b_ref[...]
    return pl.pallas_call(
        body, out_shape=jax.ShapeDtypeStruct((N_NODES, D), jnp.float32),
    )(partials, degp_t, b2)


def kernel(features, edge_index, W1, b1, W2, b2):
    src = edge_index[0].astype(jnp.int32)
    dst = edge_index[1].astype(jnp.int32)
    b1r = b1.reshape(1, D)
    b2r = b2.reshape(1, D)

    src_p, dst_p = _pad_edges(src, dst)
    degp = _sc_degrees(src.reshape(NW, EDGES_PER_W),
                       dst.reshape(NW, EDGES_PER_W))
    degp_t = jnp.transpose(degp.reshape(2 * NW, N_PAD)[:, :N_NODES])
    h1 = _tc_matmul_scale(features, W1, degp_t)
    p1 = _sc_message_pass(h1, src_p, dst_p)
    h2 = _tc_mid(p1, degp_t, b1r, W2)
    p2 = _sc_message_pass(h2, src_p, dst_p)
    return _tc_final(p2, degp_t, b2r)
